# Initial kernel scaffold; baseline (speedup 1.0000x reference)
import jax
import jax.numpy as jnp
from jax.experimental import pallas as pl


def kernel(x, edge_index, edge_weight, W, bias):
    raise NotImplementedError("write your pallas kernel here")


# speedup vs baseline: 6.3329x; 6.3329x over previous
import functools

import jax
import jax.numpy as jnp
from jax import lax
from jax.experimental import pallas as pl
from jax.experimental.pallas import tpu as pltpu
from jax.experimental.pallas import tpu_sc as plsc

NC = 2
NS = 16
L = 16
NW = NC * NS
CH = 128


def _sc_mesh():
    return plsc.VectorSubcoreMesh(
        core_axis_name="c", subcore_axis_name="s",
        num_cores=NC, num_subcores=NS)


_SC_PARAMS = pltpu.CompilerParams(needs_layout_passes=False)


def _deg_kernel(n_pad, nch):
    rps = n_pad // NS

    @functools.partial(
        pl.kernel,
        out_type=jax.ShapeDtypeStruct((NC, n_pad), jnp.float32),
        mesh=_sc_mesh(),
        compiler_params=_SC_PARAMS,
        scratch_types=[
            pltpu.VMEM((nch, CH), jnp.int32),
            pltpu.VMEM((nch, CH), jnp.float32),
            pltpu.VMEM((rps,), jnp.float32),
            pltpu.VMEM_SHARED((n_pad,), jnp.float32),
        ],
    )
    def k(row_hbm, ew_hbm, out_hbm, row_t, ew_t, zb, acc):
        c = lax.axis_index("c")
        s = lax.axis_index("s")
        wid = c * NS + s
        pltpu.sync_copy(row_hbm.at[wid], row_t)
        pltpu.sync_copy(ew_hbm.at[wid], ew_t)

        zf = jnp.zeros((L,), jnp.float32)

        @pl.loop(0, rps // L)
        def _(i):
            zb[pl.ds(i * L, L)] = zf

        pltpu.sync_copy(zb, acc.at[pl.ds(s * rps, rps)])
        plsc.subcore_barrier()

        @pl.loop(0, nch)
        def _(g):
            pltpu.sync_copy(ew_t.at[g], acc.at[row_t.at[g]], add=True)

        plsc.subcore_barrier()
        sl = pl.ds(s * rps, rps)
        pltpu.sync_copy(acc.at[sl], out_hbm.at[c, sl])

    return k


GB = 8


def _spmm_kernel(n_pad, f, nch):
    rps = n_pad // NS
    zr = CH
    assert rps % zr == 0 and nch % GB == 0

    @functools.partial(
        pl.kernel,
        out_type=jax.ShapeDtypeStruct((NC, n_pad, f), jnp.float32),
        mesh=_sc_mesh(),
        compiler_params=_SC_PARAMS,
        scratch_types=[
            pltpu.VMEM((n_pad,), jnp.float32),
            pltpu.VMEM((GB, CH), jnp.int32),
            pltpu.VMEM((GB, CH), jnp.int32),
            pltpu.VMEM((GB, CH), jnp.float32),
            pltpu.VMEM((CH, f), jnp.float32),
            pltpu.VMEM((CH,), jnp.float32),
            pltpu.VMEM_SHARED((n_pad, f), jnp.float32),
            pltpu.SemaphoreType.DMA,
        ],
    )
    def k(z_hbm, dis_hbm, src_hbm, dst_hbm, ew_hbm, out_hbm,
          dis_t, src_g, dst_g, ew_g, rows, wbuf, acc, sem):
        c = lax.axis_index("c")
        s = lax.axis_index("s")
        wid = c * NS + s
        pltpu.sync_copy(dis_hbm, dis_t)

        zf = jnp.zeros((L,), jnp.float32)

        @pl.loop(0, zr)
        def _(i):
            for q in range(f // L):
                rows[i, pl.ds(q * L, L)] = zf

        for j in range(rps // zr):
            pltpu.sync_copy(rows, acc.at[pl.ds(s * rps + j * zr, zr)])
        plsc.subcore_barrier()

        zero16 = jnp.zeros((L,), jnp.int32)

        @pl.loop(0, nch // GB)
        def _(t):
            gsl = pl.ds(t * GB, GB)
            pltpu.sync_copy(src_hbm.at[wid, gsl], src_g)
            pltpu.sync_copy(dst_hbm.at[wid, gsl], dst_g)
            pltpu.sync_copy(ew_hbm.at[wid, gsl], ew_g)

            @pl.loop(0, GB)
            def _(g):
                gather = pltpu.async_copy(z_hbm.at[src_g.at[g]], rows, sem)
                for j in range(CH // L):
                    sv = src_g[g, pl.ds(j * L, L)]
                    dv = dst_g[g, pl.ds(j * L, L)]
                    wv = ew_g[g, pl.ds(j * L, L)]
                    a = plsc.load_gather(dis_t, [sv])
                    b = plsc.load_gather(dis_t, [dv])
                    wbuf[pl.ds(j * L, L)] = -(a * wv * b)
                gather.wait()

                @pl.loop(0, CH)
                def _(i):
                    wspl = plsc.load_gather(wbuf, [zero16 + i])
                    for q in range(f // L):
                        rows[i, pl.ds(q * L, L)] = rows[i, pl.ds(q * L, L)] * wspl

                pltpu.sync_copy(rows, acc.at[dst_g.at[g]], add=True)

        plsc.subcore_barrier()
        for j in range(rps // zr):
            sl = pl.ds(s * rps + j * zr, zr)
            pltpu.sync_copy(acc.at[sl], out_hbm.at[c, sl])

    return k


def _dis_tc(deg_p):
    def body(dp_ref, o_ref):
        d = dp_ref[0] + dp_ref[1]
        o_ref[...] = jnp.where(d > 0, lax.rsqrt(d), 0.0)

    shape = deg_p.shape[1:]
    return pl.pallas_call(
        body, out_shape=jax.ShapeDtypeStruct(shape, jnp.float32))(deg_p)


def _combine1_tc(x_pad, p, w0, w1):
    n_pad, f = x_pad.shape
    bn = 640
    nb = n_pad // bn

    def body(x_ref, p_ref, w0_ref, w1_ref, t1_ref, acc_ref):
        t1 = p_ref[0] + p_ref[1]
        t1_ref[...] = t1
        acc_ref[...] = (
            jnp.dot(x_ref[...], w0_ref[...], preferred_element_type=jnp.float32)
            + jnp.dot(t1, w1_ref[...], preferred_element_type=jnp.float32))

    return pl.pallas_call(
        body,
        grid=(nb,),
        in_specs=[
            pl.BlockSpec((bn, f), lambda i: (i, 0)),
            pl.BlockSpec((NC, bn, f), lambda i: (0, i, 0)),
            pl.BlockSpec((f, f), lambda i: (0, 0)),
            pl.BlockSpec((f, f), lambda i: (0, 0)),
        ],
        out_specs=[pl.BlockSpec((bn, f), lambda i: (i, 0))] * 2,
        out_shape=[jax.ShapeDtypeStruct((n_pad, f), jnp.float32)] * 2,
    )(x_pad, p, w0, w1)


def _combine2_tc(n, x_pad, q, acc, w2, bias):
    n_pad, f = x_pad.shape
    bn = 400
    nb = n // bn
    assert nb * bn == n

    def body(x_ref, q_ref, a_ref, w2_ref, b_ref, o_ref):
        t2 = 2.0 * (q_ref[0] + q_ref[1]) - x_ref[...]
        o_ref[...] = (
            a_ref[...]
            + jnp.dot(t2, w2_ref[...], preferred_element_type=jnp.float32)
            + b_ref[...])

    return pl.pallas_call(
        body,
        grid=(nb,),
        in_specs=[
            pl.BlockSpec((bn, f), lambda i: (i, 0)),
            pl.BlockSpec((NC, bn, f), lambda i: (0, i, 0)),
            pl.BlockSpec((bn, f), lambda i: (i, 0)),
            pl.BlockSpec((f, f), lambda i: (0, 0)),
            pl.BlockSpec((1, f), lambda i: (0, 0)),
        ],
        out_specs=pl.BlockSpec((bn, f), lambda i: (i, 0)),
        out_shape=jax.ShapeDtypeStruct((n, f), jnp.float32),
    )(x_pad, q, acc, w2, bias)


def kernel(x, edge_index, edge_weight, W, bias):
    n, f = x.shape
    e = edge_weight.shape[0]
    assert W.shape[0] == 3 and f % L == 0

    n_pad = ((n + NW * L - 1) // (NW * L)) * (NW * L)
    nch = -(-e // (NW * CH * GB)) * GB
    e_pad = NW * nch * CH
    pad_e = e_pad - e

    row = jnp.pad(edge_index[0], (0, pad_e)).reshape(NW, nch, CH)
    col = jnp.pad(edge_index[1], (0, pad_e)).reshape(NW, nch, CH)
    ew = jnp.pad(edge_weight, (0, pad_e)).reshape(NW, nch, CH)
    x_pad = jnp.pad(x, ((0, n_pad - n), (0, 0)))

    deg_p = _deg_kernel(n_pad, nch)(row, ew)
    dis = _dis_tc(deg_p.reshape(NC, n_pad // f, f)).reshape(n_pad)

    spmm = _spmm_kernel(n_pad, f, nch)
    p = spmm(x_pad, dis, row, col, ew)
    tx1, acc = _combine1_tc(x_pad, p, W[0], W[1])
    q = spmm(tx1, dis, row, col, ew)
    return _combine2_tc(n, x_pad, q, acc, W[2], bias.reshape(1, f))
